# baseline (speedup 1.0000x reference)
import functools

import jax
import jax.numpy as jnp
from jax import lax
from jax.experimental import pallas as pl
from jax.experimental.pallas import tpu as pltpu
from jax.experimental.pallas import tpu_sc as plsc

N = 10000
E = 320000
G = 16
B = 400
NG = 25

NPAD = 10240
NC, NS = 2, 16
NW = NC * NS
EPW = E // NW
R = 80
ROWS_PW = EPW // R
CR = 5
CHUNK = CR * R
NCHUNK = ROWS_PW // CR
NZ = NPAD // NS


def _pre(x_ref, xg_ref, a_ref, wm_ref, w1row_ref, woutx_ref, b1row_ref,
         woutrow_ref, uw_ref, ind_ref):
    x = x_ref[:]
    dn = (((1,), (1,)), ((), ()))
    uw_ref[:, :G] = lax.dot_general(x, a_ref[:], dn,
                                    preferred_element_type=jnp.float32)
    uw_ref[:, G:] = lax.dot_general(x, wm_ref[:], dn,
                                    preferred_element_type=jnp.float32)
    xg = xg_ref[:]
    wc = w1row_ref[:] * woutx_ref[:]
    ssum = jnp.sum(xg * wc, axis=1)
    bt = jnp.sum(b1row_ref[:] * woutrow_ref[:])
    ind_ref[0, :] = ssum + bt


def _sc_body(x_hbm, uw_hbm, eidx_hbm, out_hbm,
             idx0_v, idx1_v, xd0_v, xd1_v, uw0_v, uw1_v,
             exs_v, evw_v, zbuf_v, den_sh, num_sh, sem0, sem1):
    cid = lax.axis_index("c")
    sid = lax.axis_index("s")
    wid = cid * NS + sid

    idx_b = (idx0_v, idx1_v)
    xd_b = (xd0_v, xd1_v)
    uw_b = (uw0_v, uw1_v)
    sem_b = (sem0, sem1)

    for i in range(NZ // 16):
        zbuf_v[pl.ds(i * 16, 16)] = jnp.zeros((16,), jnp.float32)
    pltpu.sync_copy(zbuf_v, den_sh.at[pl.ds(sid * NZ, NZ)])
    pltpu.sync_copy(zbuf_v, num_sh.at[pl.ds(sid * NZ, NZ)])
    plsc.subcore_barrier()

    def fire(k, p):
        rb = wid * ROWS_PW + k * CR
        pltpu.sync_copy(eidx_hbm.at[pl.ds(rb, CR)], idx_b[p])
        for r in range(CR):
            rows = pl.ds(r * R, R)
            pltpu.async_copy(uw_hbm.at[idx_b[p].at[r, 0]],
                             uw_b[p].at[rows], sem_b[p])
            pltpu.async_copy(x_hbm.at[idx_b[p].at[r, 1]],
                             xd_b[p].at[rows], sem_b[p])

    def drain(p):
        for r in range(CR):
            rows = pl.ds(r * R, R)
            pltpu.make_async_copy(uw_hbm.at[idx_b[p].at[r, 0]],
                                  uw_b[p].at[rows], sem_b[p]).wait()
            pltpu.make_async_copy(x_hbm.at[idx_b[p].at[r, 1]],
                                  xd_b[p].at[rows], sem_b[p]).wait()

    base_iota = lax.iota(jnp.int32, 16)

    def process(k, p, fire_next):
        if fire_next:
            fire(k + 1, 1 - p)
        drain(p)
        xd_v, uw_v = xd_b[p], uw_b[p]
        for g in range(CHUNK // 16):
            eidx = base_iota + (g * 16)
            s = jnp.zeros((16,), jnp.float32)
            t = jnp.zeros((16,), jnp.float32)
            for j in range(G):
                cj = jnp.full((16,), j, jnp.int32)
                xc = plsc.load_gather(xd_v, [eidx, cj])
                uc = plsc.load_gather(uw_v, [eidx, cj])
                wc = plsc.load_gather(uw_v, [eidx, cj + G])
                s = s + xc * uc
                t = t + xc * wc
            s = jnp.where(s >= 0.0, s, 0.2 * s)
            ex = jnp.exp(s)
            row, off = g // (R // 16), (g % (R // 16)) * 16
            exs_v[row, pl.ds(off, 16)] = ex
            evw_v[row, pl.ds(off, 16)] = ex * t
        for r in range(CR):
            pltpu.sync_copy(exs_v.at[r], den_sh.at[idx_b[p].at[r, 1]], add=True)
            pltpu.sync_copy(evw_v.at[r], num_sh.at[idx_b[p].at[r, 1]], add=True)

    fire(0, 0)

    def pair_body(kk, carry):
        k = kk * 2
        process(k, 0, True)
        process(k + 1, 1, True)
        return carry

    lax.fori_loop(0, (NCHUNK - 1) // 2, pair_body, 0)
    process(NCHUNK - 1, 0, False)
    plsc.subcore_barrier()

    myrows = pl.ds(sid * NZ, NZ)
    pltpu.sync_copy(den_sh.at[myrows], zbuf_v)
    pltpu.sync_copy(zbuf_v, out_hbm.at[2 * cid, myrows])
    pltpu.sync_copy(num_sh.at[myrows], zbuf_v)
    pltpu.sync_copy(zbuf_v, out_hbm.at[2 * cid + 1, myrows])


@functools.cache
def _sc_edges():
    return pl.kernel(
        _sc_body,
        out_type=jax.ShapeDtypeStruct((4, NPAD), jnp.float32),
        mesh=plsc.VectorSubcoreMesh(core_axis_name="c", subcore_axis_name="s",
                                    num_cores=NC, num_subcores=NS),
        compiler_params=pltpu.CompilerParams(use_tc_tiling_on_sc=False,
                                             needs_layout_passes=False),
        scratch_types=[
            pltpu.VMEM((CR, 2, R), jnp.int32),
            pltpu.VMEM((CR, 2, R), jnp.int32),
            pltpu.VMEM((CHUNK, G), jnp.float32),
            pltpu.VMEM((CHUNK, G), jnp.float32),
            pltpu.VMEM((CHUNK, 2 * G), jnp.float32),
            pltpu.VMEM((CHUNK, 2 * G), jnp.float32),
            pltpu.VMEM((CR, R), jnp.float32),
            pltpu.VMEM((CR, R), jnp.float32),
            pltpu.VMEM((NZ,), jnp.float32),
            pltpu.VMEM_SHARED((NPAD,), jnp.float32),
            pltpu.VMEM_SHARED((NPAD,), jnp.float32),
            pltpu.SemaphoreType.DMA,
            pltpu.SemaphoreType.DMA,
        ],
    )


def _post(part_ref, batch_ref, ind_ref, out_ref):
    den = part_ref[0, :N] + part_ref[2, :N]
    num = part_ref[1, :N] + part_ref[3, :N]
    upd = (num / (den + 1e-16))[None, :]
    batch_row = batch_ref[:]
    gb = 100
    for c in range(B // gb):
        gids = lax.broadcasted_iota(jnp.int32, (gb, N), 0) + c * gb
        mask = (batch_row == gids).astype(jnp.float32)
        sums = jnp.sum(mask * upd, axis=1)
        counts = jnp.sum(mask, axis=1)
        emb = jnp.where(counts > 0, sums / jnp.maximum(counts, 1.0), 0.0)
        out_ref[0, pl.ds(c * gb, gb)] = emb + ind_ref[0, pl.ds(c * gb, gb)]


def kernel(x, edge_index, batch, W1, b1, Wout, a, wlin):
    amat = a.reshape(G, G)
    wmat = wlin.reshape(G, G)
    xg = x.reshape(B, NG * G)
    w1row = W1.reshape(1, NG * G)
    woutx = jnp.repeat(Wout, G).reshape(1, NG * G)
    b1row = b1.reshape(1, NG)
    woutrow = Wout.reshape(1, NG)

    uw, ind = pl.pallas_call(
        _pre,
        out_shape=(
            jax.ShapeDtypeStruct((N, 2 * G), jnp.float32),
            jax.ShapeDtypeStruct((1, B), jnp.float32),
        ),
    )(x, xg, amat, wmat, w1row, woutx, b1row, woutrow)

    eidx2 = jnp.stack([edge_index[0].reshape(E // R, R),
                       edge_index[1].reshape(E // R, R)], axis=1)
    part = _sc_edges()(x, uw, eidx2)

    out = pl.pallas_call(
        _post,
        out_shape=jax.ShapeDtypeStruct((1, B), jnp.float32),
    )(part, batch.reshape(1, N), ind)
    return out.reshape(B, 1)


# speedup vs baseline: 1.7357x; 1.7357x over previous
import functools

import jax
import jax.numpy as jnp
from jax import lax
from jax.experimental import pallas as pl
from jax.experimental.pallas import tpu as pltpu
from jax.experimental.pallas import tpu_sc as plsc

N = 10000
E = 320000
G = 16
B = 400
NG = 25

NPAD = 10240
NC, NS = 2, 16
NW = NC * NS
EPW = E // NW
R = 80
ROWS_PW = EPW // R
CR = 5
CHUNK = CR * R
NCHUNK = ROWS_PW // CR
NZ = NPAD // NS


def _pre(x_ref, xg_ref, a_ref, wm_ref, w1row_ref, woutx_ref, b1row_ref,
         woutrow_ref, u_ref, w_ref, ind_ref):
    x = x_ref[:]
    dn = (((1,), (1,)), ((), ()))
    u_ref[:] = lax.dot_general(x, a_ref[:], dn,
                               preferred_element_type=jnp.float32)
    w_ref[:] = lax.dot_general(x, wm_ref[:], dn,
                               preferred_element_type=jnp.float32)
    xg = xg_ref[:]
    wc = w1row_ref[:] * woutx_ref[:]
    ssum = jnp.sum(xg * wc, axis=1)
    bt = jnp.sum(b1row_ref[:] * woutrow_ref[:])
    ind_ref[0, :] = ssum + bt


def _sc_body(x_hbm, u_hbm, w_hbm, src_hbm, dst_hbm, out_hbm,
             src_v, dst_v, xd_v, us_v, ws_v, exs_v, evw_v, zbuf_v,
             den_sh, num_sh, sem):
    cid = lax.axis_index("c")
    sid = lax.axis_index("s")
    wid = cid * NS + sid

    for i in range(NZ // 16):
        zbuf_v[pl.ds(i * 16, 16)] = jnp.zeros((16,), jnp.float32)
    pltpu.sync_copy(zbuf_v, den_sh.at[pl.ds(sid * NZ, NZ)])
    pltpu.sync_copy(zbuf_v, num_sh.at[pl.ds(sid * NZ, NZ)])
    plsc.subcore_barrier()

    def chunk_body(k, carry):
        rb = wid * ROWS_PW + k * CR
        pltpu.sync_copy(src_hbm.at[pl.ds(rb, CR)], src_v)
        pltpu.sync_copy(dst_hbm.at[pl.ds(rb, CR)], dst_v)
        descs = []
        for r in range(CR):
            rows = pl.ds(r * R, R)
            descs.append(pltpu.async_copy(x_hbm.at[dst_v.at[r]],
                                          xd_v.at[rows], sem))
            descs.append(pltpu.async_copy(u_hbm.at[src_v.at[r]],
                                          us_v.at[rows], sem))
            descs.append(pltpu.async_copy(w_hbm.at[src_v.at[r]],
                                          ws_v.at[rows], sem))
        for d in descs:
            d.wait()
        base_iota = lax.iota(jnp.int32, 16)
        diags = [(base_iota + d) & (G - 1) for d in range(G)]

        def group_body(g, c2):
            eidx = base_iota + g * 16
            s = jnp.zeros((16,), jnp.float32)
            t = jnp.zeros((16,), jnp.float32)
            for d in range(G):
                cj = diags[d]
                xc = plsc.load_gather(xd_v, [eidx, cj])
                uc = plsc.load_gather(us_v, [eidx, cj])
                wc = plsc.load_gather(ws_v, [eidx, cj])
                s = s + xc * uc
                t = t + xc * wc
            s = jnp.where(s >= 0.0, s, 0.2 * s)
            ex = jnp.exp(s)
            exs_v[pl.ds(g * 16, 16)] = ex
            evw_v[pl.ds(g * 16, 16)] = ex * t
            return c2

        lax.fori_loop(0, CHUNK // 16, group_body, 0)
        for r in range(CR):
            rows = pl.ds(r * R, R)
            pltpu.sync_copy(exs_v.at[rows], den_sh.at[dst_v.at[r]], add=True)
            pltpu.sync_copy(evw_v.at[rows], num_sh.at[dst_v.at[r]], add=True)
        return carry

    lax.fori_loop(0, NCHUNK, chunk_body, 0)
    plsc.subcore_barrier()

    myrows = pl.ds(sid * NZ, NZ)
    pltpu.sync_copy(den_sh.at[myrows], zbuf_v)
    pltpu.sync_copy(zbuf_v, out_hbm.at[2 * cid, myrows])
    pltpu.sync_copy(num_sh.at[myrows], zbuf_v)
    pltpu.sync_copy(zbuf_v, out_hbm.at[2 * cid + 1, myrows])


@functools.cache
def _sc_edges():
    return pl.kernel(
        _sc_body,
        out_type=jax.ShapeDtypeStruct((4, NPAD), jnp.float32),
        mesh=plsc.VectorSubcoreMesh(core_axis_name="c", subcore_axis_name="s",
                                    num_cores=NC, num_subcores=NS),
        compiler_params=pltpu.CompilerParams(use_tc_tiling_on_sc=False,
                                             needs_layout_passes=False),
        scratch_types=[
            pltpu.VMEM((CR, R), jnp.int32),
            pltpu.VMEM((CR, R), jnp.int32),
            pltpu.VMEM((CHUNK, G), jnp.float32),
            pltpu.VMEM((CHUNK, G), jnp.float32),
            pltpu.VMEM((CHUNK, G), jnp.float32),
            pltpu.VMEM((CHUNK,), jnp.float32),
            pltpu.VMEM((CHUNK,), jnp.float32),
            pltpu.VMEM((NZ,), jnp.float32),
            pltpu.VMEM_SHARED((NPAD,), jnp.float32),
            pltpu.VMEM_SHARED((NPAD,), jnp.float32),
            pltpu.SemaphoreType.DMA,
        ],
    )


def _post(part_ref, batch_ref, ind_ref, out_ref):
    den = part_ref[0, :N] + part_ref[2, :N]
    num = part_ref[1, :N] + part_ref[3, :N]
    upd = (num / (den + 1e-16))[None, :]
    batch_row = batch_ref[:]
    gb = 100
    for c in range(B // gb):
        gids = lax.broadcasted_iota(jnp.int32, (gb, N), 0) + c * gb
        mask = (batch_row == gids).astype(jnp.float32)
        sums = jnp.sum(mask * upd, axis=1)
        counts = jnp.sum(mask, axis=1)
        emb = jnp.where(counts > 0, sums / jnp.maximum(counts, 1.0), 0.0)
        out_ref[0, pl.ds(c * gb, gb)] = emb + ind_ref[0, pl.ds(c * gb, gb)]


def kernel(x, edge_index, batch, W1, b1, Wout, a, wlin):
    amat = a.reshape(G, G)
    wmat = wlin.reshape(G, G)
    xg = x.reshape(B, NG * G)
    w1row = W1.reshape(1, NG * G)
    woutx = jnp.repeat(Wout, G).reshape(1, NG * G)
    b1row = b1.reshape(1, NG)
    woutrow = Wout.reshape(1, NG)

    u, w, ind = pl.pallas_call(
        _pre,
        out_shape=(
            jax.ShapeDtypeStruct((N, G), jnp.float32),
            jax.ShapeDtypeStruct((N, G), jnp.float32),
            jax.ShapeDtypeStruct((1, B), jnp.float32),
        ),
    )(x, xg, amat, wmat, w1row, woutx, b1row, woutrow)

    src2d = edge_index[0].reshape(E // R, R)
    dst2d = edge_index[1].reshape(E // R, R)
    part = _sc_edges()(x, u, w, src2d, dst2d)

    out = pl.pallas_call(
        _post,
        out_shape=jax.ShapeDtypeStruct((1, B), jnp.float32),
    )(part, batch.reshape(1, N), ind)
    return out.reshape(B, 1)


# speedup vs baseline: 2.0937x; 1.2062x over previous
import functools

import jax
import jax.numpy as jnp
from jax import lax
from jax.experimental import pallas as pl
from jax.experimental.pallas import tpu as pltpu
from jax.experimental.pallas import tpu_sc as plsc

N = 10000
E = 320000
G = 16
B = 400
NG = 25

NPAD = 10240
NC, NS = 2, 16
NW = NC * NS
EPW = E // NW
R = 80
ROWS_PW = EPW // R
CR = 25
CHUNK = CR * R
NCHUNK = ROWS_PW // CR
NZ = NPAD // NS


def _pre(x_ref, xg_ref, a_ref, wm_ref, w1row_ref, woutx_ref, b1row_ref,
         woutrow_ref, u_ref, w_ref, ind_ref):
    x = x_ref[:]
    dn = (((1,), (1,)), ((), ()))
    u_ref[:] = lax.dot_general(x, a_ref[:], dn,
                               preferred_element_type=jnp.float32)
    w_ref[:] = lax.dot_general(x, wm_ref[:], dn,
                               preferred_element_type=jnp.float32)
    xg = xg_ref[:]
    wc = w1row_ref[:] * woutx_ref[:]
    ssum = jnp.sum(xg * wc, axis=1)
    bt = jnp.sum(b1row_ref[:] * woutrow_ref[:])
    ind_ref[0, :] = ssum + bt


def _sc_body(x_hbm, u_hbm, w_hbm, src_hbm, dst_hbm, out_hbm,
             src_v, dst_v, xd_v, us_v, ws_v, exs_v, evw_v, zbuf_v,
             den_sh, num_sh, sem):
    cid = lax.axis_index("c")
    sid = lax.axis_index("s")
    wid = cid * NS + sid

    for i in range(NZ // 16):
        zbuf_v[pl.ds(i * 16, 16)] = jnp.zeros((16,), jnp.float32)
    pltpu.sync_copy(zbuf_v, den_sh.at[pl.ds(sid * NZ, NZ)])
    pltpu.sync_copy(zbuf_v, num_sh.at[pl.ds(sid * NZ, NZ)])
    plsc.subcore_barrier()

    def chunk_body(k, carry):
        rb = wid * ROWS_PW + k * CR
        pltpu.sync_copy(src_hbm.at[pl.ds(rb, CR)], src_v)
        pltpu.sync_copy(dst_hbm.at[pl.ds(rb, CR)], dst_v)
        descs = []
        for r in range(CR):
            rows = pl.ds(r * R, R)
            descs.append(pltpu.async_copy(x_hbm.at[dst_v.at[r]],
                                          xd_v.at[rows], sem))
            descs.append(pltpu.async_copy(u_hbm.at[src_v.at[r]],
                                          us_v.at[rows], sem))
            descs.append(pltpu.async_copy(w_hbm.at[src_v.at[r]],
                                          ws_v.at[rows], sem))
        for d in descs:
            d.wait()
        base_iota = lax.iota(jnp.int32, 16)
        diags = [(base_iota + d) & (G - 1) for d in range(G)]

        def group_body(g, c2):
            eidx = base_iota + g * 16
            s = jnp.zeros((16,), jnp.float32)
            t = jnp.zeros((16,), jnp.float32)
            for d in range(G):
                cj = diags[d]
                xc = plsc.load_gather(xd_v, [eidx, cj])
                uc = plsc.load_gather(us_v, [eidx, cj])
                wc = plsc.load_gather(ws_v, [eidx, cj])
                s = s + xc * uc
                t = t + xc * wc
            s = jnp.where(s >= 0.0, s, 0.2 * s)
            ex = jnp.exp(s)
            exs_v[pl.ds(g * 16, 16)] = ex
            evw_v[pl.ds(g * 16, 16)] = ex * t
            return c2

        lax.fori_loop(0, CHUNK // 16, group_body, 0)
        for r in range(CR):
            rows = pl.ds(r * R, R)
            pltpu.sync_copy(exs_v.at[rows], den_sh.at[dst_v.at[r]], add=True)
            pltpu.sync_copy(evw_v.at[rows], num_sh.at[dst_v.at[r]], add=True)
        return carry

    lax.fori_loop(0, NCHUNK, chunk_body, 0)
    plsc.subcore_barrier()

    myrows = pl.ds(sid * NZ, NZ)
    pltpu.sync_copy(den_sh.at[myrows], zbuf_v)
    pltpu.sync_copy(zbuf_v, out_hbm.at[2 * cid, myrows])
    pltpu.sync_copy(num_sh.at[myrows], zbuf_v)
    pltpu.sync_copy(zbuf_v, out_hbm.at[2 * cid + 1, myrows])


@functools.cache
def _sc_edges():
    return pl.kernel(
        _sc_body,
        out_type=jax.ShapeDtypeStruct((4, NPAD), jnp.float32),
        mesh=plsc.VectorSubcoreMesh(core_axis_name="c", subcore_axis_name="s",
                                    num_cores=NC, num_subcores=NS),
        compiler_params=pltpu.CompilerParams(use_tc_tiling_on_sc=False,
                                             needs_layout_passes=False),
        scratch_types=[
            pltpu.VMEM((CR, R), jnp.int32),
            pltpu.VMEM((CR, R), jnp.int32),
            pltpu.VMEM((CHUNK, G), jnp.float32),
            pltpu.VMEM((CHUNK, G), jnp.float32),
            pltpu.VMEM((CHUNK, G), jnp.float32),
            pltpu.VMEM((CHUNK,), jnp.float32),
            pltpu.VMEM((CHUNK,), jnp.float32),
            pltpu.VMEM((NZ,), jnp.float32),
            pltpu.VMEM_SHARED((NPAD,), jnp.float32),
            pltpu.VMEM_SHARED((NPAD,), jnp.float32),
            pltpu.SemaphoreType.DMA,
        ],
    )


def _post(part_ref, batch_ref, ind_ref, out_ref):
    den = part_ref[0, :N] + part_ref[2, :N]
    num = part_ref[1, :N] + part_ref[3, :N]
    upd = (num / (den + 1e-16))[None, :]
    batch_row = batch_ref[:]
    gb = 100
    for c in range(B // gb):
        gids = lax.broadcasted_iota(jnp.int32, (gb, N), 0) + c * gb
        mask = (batch_row == gids).astype(jnp.float32)
        sums = jnp.sum(mask * upd, axis=1)
        counts = jnp.sum(mask, axis=1)
        emb = jnp.where(counts > 0, sums / jnp.maximum(counts, 1.0), 0.0)
        out_ref[0, pl.ds(c * gb, gb)] = emb + ind_ref[0, pl.ds(c * gb, gb)]


def kernel(x, edge_index, batch, W1, b1, Wout, a, wlin):
    amat = a.reshape(G, G)
    wmat = wlin.reshape(G, G)
    xg = x.reshape(B, NG * G)
    w1row = W1.reshape(1, NG * G)
    woutx = jnp.repeat(Wout, G).reshape(1, NG * G)
    b1row = b1.reshape(1, NG)
    woutrow = Wout.reshape(1, NG)

    u, w, ind = pl.pallas_call(
        _pre,
        out_shape=(
            jax.ShapeDtypeStruct((N, G), jnp.float32),
            jax.ShapeDtypeStruct((N, G), jnp.float32),
            jax.ShapeDtypeStruct((1, B), jnp.float32),
        ),
    )(x, xg, amat, wmat, w1row, woutx, b1row, woutrow)

    src2d = edge_index[0].reshape(E // R, R)
    dst2d = edge_index[1].reshape(E // R, R)
    part = _sc_edges()(x, u, w, src2d, dst2d)

    out = pl.pallas_call(
        _post,
        out_shape=jax.ShapeDtypeStruct((1, B), jnp.float32),
    )(part, batch.reshape(1, N), ind)
    return out.reshape(B, 1)


# speedup vs baseline: 2.2717x; 1.0850x over previous
import functools

import jax
import jax.numpy as jnp
from jax import lax
from jax.experimental import pallas as pl
from jax.experimental.pallas import tpu as pltpu
from jax.experimental.pallas import tpu_sc as plsc

N = 10000
E = 320000
G = 16
B = 400
NG = 25

NPAD = 10240
NC, NS = 2, 16
NW = NC * NS
EPW = E // NW
R = 80
ROWS_PW = EPW // R
CR = 25
CHUNK = CR * R
NCHUNK = ROWS_PW // CR
NZ = NPAD // NS


def _pre(x_ref, xg_ref, a_ref, wm_ref, w1row_ref, woutx_ref, b1row_ref,
         woutrow_ref, u_ref, w_ref, ind_ref):
    x = x_ref[:]
    dn = (((1,), (1,)), ((), ()))
    u_ref[:] = lax.dot_general(x, a_ref[:], dn,
                               preferred_element_type=jnp.float32)
    w_ref[:] = lax.dot_general(x, wm_ref[:], dn,
                               preferred_element_type=jnp.float32)
    xg = xg_ref[:]
    wc = w1row_ref[:] * woutx_ref[:]
    ssum = jnp.sum(xg * wc, axis=1)
    bt = jnp.sum(b1row_ref[:] * woutrow_ref[:])
    ind_ref[0, :] = ssum + bt


def _sc_body(x_hbm, u_hbm, w_hbm, src_hbm, dst_hbm, out_hbm,
             src_v, dst_v, xd_v, us_v, ws_v, exs_v, evw_v, zbuf_v,
             den_sh, num_sh, sem, sem2):
    cid = lax.axis_index("c")
    sid = lax.axis_index("s")
    wid = cid * NS + sid

    for i in range(NZ // 16):
        zbuf_v[pl.ds(i * 16, 16)] = jnp.zeros((16,), jnp.float32)
    pltpu.sync_copy(zbuf_v, den_sh.at[pl.ds(sid * NZ, NZ)])
    pltpu.sync_copy(zbuf_v, num_sh.at[pl.ds(sid * NZ, NZ)])
    plsc.subcore_barrier()

    base_iota = lax.iota(jnp.int32, 16)
    diags = [(base_iota + d) & (G - 1) for d in range(G)]

    def chunk_body(k, carry):
        rb = wid * ROWS_PW + k * CR
        pltpu.sync_copy(src_hbm.at[pl.ds(rb, CR)], src_v)
        pltpu.sync_copy(dst_hbm.at[pl.ds(rb, CR)], dst_v)
        gd = []
        for r in range(CR):
            rows = pl.ds(r * R, R)
            gd.append((pltpu.async_copy(x_hbm.at[dst_v.at[r]],
                                        xd_v.at[rows], sem),
                       pltpu.async_copy(u_hbm.at[src_v.at[r]],
                                        us_v.at[rows], sem),
                       pltpu.async_copy(w_hbm.at[src_v.at[r]],
                                        ws_v.at[rows], sem)))
        sd = []
        for r in range(CR):
            for d in gd[r]:
                d.wait()

            def group_body(q, c2, r=r):
                eidx = base_iota + (r * R + q * 16)
                s = jnp.zeros((16,), jnp.float32)
                t = jnp.zeros((16,), jnp.float32)
                for d in range(G):
                    cj = diags[d]
                    xc = plsc.load_gather(xd_v, [eidx, cj])
                    uc = plsc.load_gather(us_v, [eidx, cj])
                    wc = plsc.load_gather(ws_v, [eidx, cj])
                    s = s + xc * uc
                    t = t + xc * wc
                s = jnp.where(s >= 0.0, s, 0.2 * s)
                ex = jnp.exp(s)
                exs_v[pl.ds(r * R + q * 16, 16)] = ex
                evw_v[pl.ds(r * R + q * 16, 16)] = ex * t
                return c2

            lax.fori_loop(0, R // 16, group_body, 0)
            rows = pl.ds(r * R, R)
            sd.append(pltpu.async_copy(exs_v.at[rows],
                                       den_sh.at[dst_v.at[r]], sem2, add=True))
            sd.append(pltpu.async_copy(evw_v.at[rows],
                                       num_sh.at[dst_v.at[r]], sem2, add=True))
        for d in sd:
            d.wait()
        return carry

    lax.fori_loop(0, NCHUNK, chunk_body, 0)
    plsc.subcore_barrier()

    myrows = pl.ds(sid * NZ, NZ)
    pltpu.sync_copy(den_sh.at[myrows], zbuf_v)
    pltpu.sync_copy(zbuf_v, out_hbm.at[2 * cid, myrows])
    pltpu.sync_copy(num_sh.at[myrows], zbuf_v)
    pltpu.sync_copy(zbuf_v, out_hbm.at[2 * cid + 1, myrows])


@functools.cache
def _sc_edges():
    return pl.kernel(
        _sc_body,
        out_type=jax.ShapeDtypeStruct((4, NPAD), jnp.float32),
        mesh=plsc.VectorSubcoreMesh(core_axis_name="c", subcore_axis_name="s",
                                    num_cores=NC, num_subcores=NS),
        compiler_params=pltpu.CompilerParams(use_tc_tiling_on_sc=False,
                                             needs_layout_passes=False),
        scratch_types=[
            pltpu.VMEM((CR, R), jnp.int32),
            pltpu.VMEM((CR, R), jnp.int32),
            pltpu.VMEM((CHUNK, G), jnp.float32),
            pltpu.VMEM((CHUNK, G), jnp.float32),
            pltpu.VMEM((CHUNK, G), jnp.float32),
            pltpu.VMEM((CHUNK,), jnp.float32),
            pltpu.VMEM((CHUNK,), jnp.float32),
            pltpu.VMEM((NZ,), jnp.float32),
            pltpu.VMEM_SHARED((NPAD,), jnp.float32),
            pltpu.VMEM_SHARED((NPAD,), jnp.float32),
            pltpu.SemaphoreType.DMA,
            pltpu.SemaphoreType.DMA,
        ],
    )


def _post(part_ref, batch_ref, ind_ref, out_ref):
    den = part_ref[0, :N] + part_ref[2, :N]
    num = part_ref[1, :N] + part_ref[3, :N]
    upd = (num / (den + 1e-16))[None, :]
    batch_row = batch_ref[:]
    gb = 100
    for c in range(B // gb):
        gids = lax.broadcasted_iota(jnp.int32, (gb, N), 0) + c * gb
        mask = (batch_row == gids).astype(jnp.float32)
        sums = jnp.sum(mask * upd, axis=1)
        counts = jnp.sum(mask, axis=1)
        emb = jnp.where(counts > 0, sums / jnp.maximum(counts, 1.0), 0.0)
        out_ref[0, pl.ds(c * gb, gb)] = emb + ind_ref[0, pl.ds(c * gb, gb)]


def kernel(x, edge_index, batch, W1, b1, Wout, a, wlin):
    amat = a.reshape(G, G)
    wmat = wlin.reshape(G, G)
    xg = x.reshape(B, NG * G)
    w1row = W1.reshape(1, NG * G)
    woutx = jnp.repeat(Wout, G).reshape(1, NG * G)
    b1row = b1.reshape(1, NG)
    woutrow = Wout.reshape(1, NG)

    u, w, ind = pl.pallas_call(
        _pre,
        out_shape=(
            jax.ShapeDtypeStruct((N, G), jnp.float32),
            jax.ShapeDtypeStruct((N, G), jnp.float32),
            jax.ShapeDtypeStruct((1, B), jnp.float32),
        ),
    )(x, xg, amat, wmat, w1row, woutx, b1row, woutrow)

    src2d = edge_index[0].reshape(E // R, R)
    dst2d = edge_index[1].reshape(E // R, R)
    part = _sc_edges()(x, u, w, src2d, dst2d)

    out = pl.pallas_call(
        _post,
        out_shape=jax.ShapeDtypeStruct((1, B), jnp.float32),
    )(part, batch.reshape(1, N), ind)
    return out.reshape(B, 1)
